# baseline (speedup 1.0000x reference)
import functools

import jax
import jax.numpy as jnp
from jax import lax
from jax.experimental import pallas as pl
from jax.experimental.pallas import tpu as pltpu
from jax.experimental.pallas import tpu_sc as plsc

B, S, E = 4, 4096, 1024
L = 16
NC, NS = 2, 16
NW = NC * NS
ROWS = B * S
RPW = ROWS // NW
C = 32
NCHUNK = RPW // C
CW = C * E


def kernel(x, pos_table):
    xf = x.reshape(ROWS * E)
    pf = pos_table.reshape(-1)

    mesh = plsc.VectorSubcoreMesh(core_axis_name="c", subcore_axis_name="s")

    @functools.partial(
        pl.kernel,
        out_type=jax.ShapeDtypeStruct((ROWS * E,), jnp.float32),
        mesh=mesh,
        scratch_types=[
            pltpu.VMEM((CW,), jnp.float32),
            pltpu.VMEM((CW,), jnp.float32),
        ],
    )
    def sc_add(x_hbm, pos_hbm, out_hbm, xbuf, pbuf):
        wid = lax.axis_index("s") * NC + lax.axis_index("c")
        base = wid * RPW
        pos_base = lax.rem(base, S)

        def chunk(i, carry):
            off = (base + i * C) * E
            poff = (pos_base + i * C) * E
            pltpu.sync_copy(x_hbm.at[pl.ds(off, CW)], xbuf)
            pltpu.sync_copy(pos_hbm.at[pl.ds(poff, CW)], pbuf)

            @plsc.parallel_loop(0, CW, step=L, unroll=8)
            def add_body(o):
                xbuf[pl.ds(o, L)] = xbuf[pl.ds(o, L)] + pbuf[pl.ds(o, L)]

            pltpu.sync_copy(xbuf, out_hbm.at[pl.ds(off, CW)])
            return carry

        lax.fori_loop(0, NCHUNK, chunk, 0)

    out = sc_add(xf, pf)
    return out.reshape(B, S, E)


# speedup vs baseline: 1.2465x; 1.2465x over previous
import functools

import jax
import jax.numpy as jnp
from jax import lax
from jax.experimental import pallas as pl
from jax.experimental.pallas import tpu as pltpu
from jax.experimental.pallas import tpu_sc as plsc

B, S, E = 4, 4096, 1024
L = 16
NC, NS = 2, 16
NW = NC * NS
PPW = S // NW
C = 16
NG = PPW // C
NT = NG * B
CW = C * E


def kernel(x, pos_table):
    xf = x.reshape(B * S * E)
    pf = pos_table.reshape(-1)

    mesh = plsc.VectorSubcoreMesh(core_axis_name="c", subcore_axis_name="s")

    @functools.partial(
        pl.kernel,
        out_type=jax.ShapeDtypeStruct((B * S * E,), jnp.float32),
        mesh=mesh,
        scratch_types=[
            pltpu.VMEM((CW,), jnp.float32),
            pltpu.VMEM((CW,), jnp.float32),
            pltpu.VMEM((CW,), jnp.float32),
            pltpu.VMEM((CW,), jnp.float32),
            pltpu.SemaphoreType.DMA,
            pltpu.SemaphoreType.DMA,
            pltpu.SemaphoreType.DMA,
            pltpu.SemaphoreType.DMA,
            pltpu.SemaphoreType.DMA,
            pltpu.SemaphoreType.DMA,
        ],
    )
    def sc_add(x_hbm, pos_hbm, out_hbm, x0, x1, p0, p1,
               sx0, sx1, sp0, sp1, so0, so1):
        wid = lax.axis_index("s") * NC + lax.axis_index("c")
        pos0 = wid * PPW

        xb = (x0, x1)
        sx = (sx0, sx1)
        pb = (p0, p1)
        sp = (sp0, sp1)
        so = (so0, so1)

        def x_off(t):
            g, b = divmod(t, B)
            return (b * S + pos0 + g * C) * E

        def p_off(g):
            return (pos0 + g * C) * E

        def start_x_in(t):
            pltpu.async_copy(x_hbm.at[pl.ds(x_off(t), CW)], xb[t % 2], sx[t % 2])

        def start_p_in(g):
            pltpu.async_copy(pos_hbm.at[pl.ds(p_off(g), CW)], pb[g % 2], sp[g % 2])

        start_p_in(0)
        start_x_in(0)

        for t in range(NT):
            g = t // B
            if t + 1 < NT:
                if t >= 1:
                    pltpu.make_async_copy(
                        xb[(t + 1) % 2],
                        out_hbm.at[pl.ds(x_off(t - 1), CW)],
                        so[(t + 1) % 2],
                    ).wait()
                start_x_in(t + 1)
            if t % B == B - 1 and g + 1 < NG:
                start_p_in(g + 1)
            pltpu.make_async_copy(
                x_hbm.at[pl.ds(x_off(t), CW)], xb[t % 2], sx[t % 2]
            ).wait()
            if t % B == 0:
                pltpu.make_async_copy(
                    pos_hbm.at[pl.ds(p_off(g), CW)], pb[g % 2], sp[g % 2]
                ).wait()

            xbuf = xb[t % 2]
            pbuf = pb[g % 2]

            @plsc.parallel_loop(0, CW, step=L, unroll=8)
            def add_body(o):
                xbuf[pl.ds(o, L)] = xbuf[pl.ds(o, L)] + pbuf[pl.ds(o, L)]

            pltpu.async_copy(xbuf, out_hbm.at[pl.ds(x_off(t), CW)], so[t % 2])

        for t in (NT - 2, NT - 1):
            pltpu.make_async_copy(
                xb[t % 2], out_hbm.at[pl.ds(x_off(t), CW)], so[t % 2]
            ).wait()

    out = sc_add(xf, pf)
    return out.reshape(B, S, E)


# speedup vs baseline: 1.2788x; 1.0259x over previous
import functools

import jax
import jax.numpy as jnp
from jax import lax
from jax.experimental import pallas as pl
from jax.experimental.pallas import tpu as pltpu
from jax.experimental.pallas import tpu_sc as plsc

B, S, E = 4, 4096, 1024
L = 16
NC, NS = 2, 16
NW = NC * NS
PPW = S // NW
C = 16
NG = PPW // C
NT = NG * B
CW = C * E


def kernel(x, pos_table):
    xf = x.reshape(B * S * E)
    pf = pos_table.reshape(-1)

    mesh = plsc.VectorSubcoreMesh(core_axis_name="c", subcore_axis_name="s")

    @functools.partial(
        pl.kernel,
        out_type=jax.ShapeDtypeStruct((B * S * E,), jnp.float32),
        mesh=mesh,
        scratch_types=[
            pltpu.VMEM((CW,), jnp.float32),
            pltpu.VMEM((CW,), jnp.float32),
            pltpu.VMEM((CW,), jnp.float32),
            pltpu.VMEM((CW,), jnp.float32),
            pltpu.VMEM((CW,), jnp.float32),
            pltpu.VMEM((CW,), jnp.float32),
            pltpu.SemaphoreType.DMA,
            pltpu.SemaphoreType.DMA,
            pltpu.SemaphoreType.DMA,
            pltpu.SemaphoreType.DMA,
            pltpu.SemaphoreType.DMA,
            pltpu.SemaphoreType.DMA,
        ],
    )
    def sc_add(x_hbm, pos_hbm, out_hbm, x0, x1, p0, p1, o0, o1,
               sx0, sx1, sp0, sp1, so0, so1):
        wid = lax.axis_index("s") * NC + lax.axis_index("c")
        pos0 = wid * PPW

        xb = (x0, x1)
        sx = (sx0, sx1)
        pb = (p0, p1)
        sp = (sp0, sp1)
        ob = (o0, o1)
        so = (so0, so1)

        def x_off(t):
            g, b = divmod(t, B)
            return (b * S + pos0 + g * C) * E

        def p_off(g):
            return (pos0 + g * C) * E

        def start_x_in(t):
            pltpu.async_copy(x_hbm.at[pl.ds(x_off(t), CW)], xb[t % 2], sx[t % 2])

        def start_p_in(g):
            pltpu.async_copy(pos_hbm.at[pl.ds(p_off(g), CW)], pb[g % 2], sp[g % 2])

        start_p_in(0)
        start_x_in(0)

        for t in range(NT):
            g = t // B
            if t + 1 < NT:
                start_x_in(t + 1)
            if t % B == B - 1 and g + 1 < NG:
                start_p_in(g + 1)
            pltpu.make_async_copy(
                x_hbm.at[pl.ds(x_off(t), CW)], xb[t % 2], sx[t % 2]
            ).wait()
            if t % B == 0:
                pltpu.make_async_copy(
                    pos_hbm.at[pl.ds(p_off(g), CW)], pb[g % 2], sp[g % 2]
                ).wait()
            if t >= 2:
                pltpu.make_async_copy(
                    ob[t % 2], out_hbm.at[pl.ds(x_off(t - 2), CW)], so[t % 2]
                ).wait()

            xbuf = xb[t % 2]
            pbuf = pb[g % 2]
            obuf = ob[t % 2]

            @plsc.parallel_loop(0, CW, step=L, unroll=8)
            def add_body(o):
                obuf[pl.ds(o, L)] = xbuf[pl.ds(o, L)] + pbuf[pl.ds(o, L)]

            pltpu.async_copy(obuf, out_hbm.at[pl.ds(x_off(t), CW)], so[t % 2])

        for t in (NT - 2, NT - 1):
            pltpu.make_async_copy(
                ob[t % 2], out_hbm.at[pl.ds(x_off(t), CW)], so[t % 2]
            ).wait()

    out = sc_add(xf, pf)
    return out.reshape(B, S, E)


# speedup vs baseline: 3.3770x; 2.6408x over previous
import functools

import jax
import jax.numpy as jnp
from jax import lax
from jax.experimental import pallas as pl
from jax.experimental.pallas import tpu as pltpu
from jax.experimental.pallas import tpu_sc as plsc

B, S, E = 4, 4096, 1024
L = 16
NC, NS = 2, 16
NW = NC * NS
PPW = S // NW
C = 16
NG = PPW // C
NT = NG * B


def kernel(x, pos_table):
    xf = x.reshape(B * S, E)

    mesh = plsc.VectorSubcoreMesh(core_axis_name="c", subcore_axis_name="s")

    @functools.partial(
        pl.kernel,
        out_type=jax.ShapeDtypeStruct((B * S, E), jnp.float32),
        mesh=mesh,
        scratch_types=[
            pltpu.VMEM((C, E), jnp.float32),
            pltpu.VMEM((C, E), jnp.float32),
            pltpu.VMEM((C, E), jnp.float32),
            pltpu.VMEM((C, E), jnp.float32),
            pltpu.VMEM((C, E), jnp.float32),
            pltpu.VMEM((C, E), jnp.float32),
            pltpu.SemaphoreType.DMA,
            pltpu.SemaphoreType.DMA,
            pltpu.SemaphoreType.DMA,
            pltpu.SemaphoreType.DMA,
            pltpu.SemaphoreType.DMA,
            pltpu.SemaphoreType.DMA,
        ],
    )
    def sc_add(x_hbm, pos_hbm, out_hbm, x0, x1, p0, p1, o0, o1,
               sx0, sx1, sp0, sp1, so0, so1):
        wid = lax.axis_index("s") * NC + lax.axis_index("c")
        pos0 = wid * PPW

        xb = (x0, x1)
        sx = (sx0, sx1)
        pb = (p0, p1)
        sp = (sp0, sp1)
        ob = (o0, o1)
        so = (so0, so1)

        def x_row(t):
            g, b = divmod(t, B)
            return b * S + pos0 + g * C

        def p_row(g):
            return pos0 + g * C

        def start_x_in(t):
            pltpu.async_copy(x_hbm.at[pl.ds(x_row(t), C)], xb[t % 2], sx[t % 2])

        def start_p_in(g):
            pltpu.async_copy(pos_hbm.at[pl.ds(p_row(g), C)], pb[g % 2], sp[g % 2])

        start_p_in(0)
        start_x_in(0)

        for t in range(NT):
            g = t // B
            if t + 1 < NT:
                start_x_in(t + 1)
            if t % B == B - 1 and g + 1 < NG:
                start_p_in(g + 1)
            pltpu.make_async_copy(
                x_hbm.at[pl.ds(x_row(t), C)], xb[t % 2], sx[t % 2]
            ).wait()
            if t % B == 0:
                pltpu.make_async_copy(
                    pos_hbm.at[pl.ds(p_row(g), C)], pb[g % 2], sp[g % 2]
                ).wait()
            if t >= 2:
                pltpu.make_async_copy(
                    ob[t % 2], out_hbm.at[pl.ds(x_row(t - 2), C)], so[t % 2]
                ).wait()

            xbuf = xb[t % 2]
            pbuf = pb[g % 2]
            obuf = ob[t % 2]

            @plsc.parallel_loop(0, E, step=L, unroll=1)
            def add_body(o):
                for r in range(C):
                    obuf[r, pl.ds(o, L)] = xbuf[r, pl.ds(o, L)] + pbuf[r, pl.ds(o, L)]

            pltpu.async_copy(obuf, out_hbm.at[pl.ds(x_row(t), C)], so[t % 2])

        for t in (NT - 2, NT - 1):
            pltpu.make_async_copy(
                ob[t % 2], out_hbm.at[pl.ds(x_row(t), C)], so[t % 2]
            ).wait()

    out = sc_add(xf, pos_table)
    return out.reshape(B, S, E)


# speedup vs baseline: 3.4769x; 1.0296x over previous
import functools

import jax
import jax.numpy as jnp
from jax import lax
from jax.experimental import pallas as pl
from jax.experimental.pallas import tpu as pltpu
from jax.experimental.pallas import tpu_sc as plsc

B, S, E = 4, 4096, 1024
L = 16
NC, NS = 2, 16
NW = NC * NS
PPW = S // NW
C = 8
NG = PPW // C
NXB = 3


def kernel(x, pos_table):
    xf = x.reshape(B * S, E)

    mesh = plsc.VectorSubcoreMesh(core_axis_name="c", subcore_axis_name="s")

    scratch = (
        [pltpu.VMEM((C, E), jnp.float32) for _ in range(B * NXB)]
        + [pltpu.VMEM((C, E), jnp.float32) for _ in range(2)]
        + [pltpu.SemaphoreType.DMA for _ in range(B * NXB + 2)]
    )

    @functools.partial(
        pl.kernel,
        out_type=jax.ShapeDtypeStruct((B * S, E), jnp.float32),
        mesh=mesh,
        scratch_types=scratch,
    )
    def sc_add(x_hbm, pos_hbm, out_hbm, *bufs):
        xb = [[bufs[b * NXB + j] for j in range(NXB)] for b in range(B)]
        pb = [bufs[B * NXB], bufs[B * NXB + 1]]
        sems = bufs[B * NXB + 2:]
        sxb = [[sems[b * NXB + j] for j in range(NXB)] for b in range(B)]
        spb = [sems[B * NXB], sems[B * NXB + 1]]

        wid = lax.axis_index("s") * NC + lax.axis_index("c")
        pos0 = wid * PPW

        def p_row(g):
            return pos0 + g * C

        def x_row(b, g):
            return b * S + pos0 + g * C

        def start_x_in(b, g):
            j = g % NXB
            pltpu.async_copy(x_hbm.at[pl.ds(x_row(b, g), C)], xb[b][j], sxb[b][j])

        def start_p_in(g):
            pltpu.async_copy(pos_hbm.at[pl.ds(p_row(g), C)], pb[g % 2], spb[g % 2])

        def wait_x_in(b, g):
            j = g % NXB
            pltpu.make_async_copy(
                x_hbm.at[pl.ds(x_row(b, g), C)], xb[b][j], sxb[b][j]
            ).wait()

        def wait_out(b, g):
            j = g % NXB
            pltpu.make_async_copy(
                xb[b][j], out_hbm.at[pl.ds(x_row(b, g), C)], sxb[b][j]
            ).wait()

        start_p_in(0)
        for b in range(B):
            start_x_in(b, 0)

        for g in range(NG):
            if g + 1 < NG:
                for b in range(B):
                    if g - 2 >= 0:
                        wait_out(b, g - 2)
                    start_x_in(b, g + 1)
                start_p_in(g + 1)
            for b in range(B):
                wait_x_in(b, g)
            pltpu.make_async_copy(
                pos_hbm.at[pl.ds(p_row(g), C)], pb[g % 2], spb[g % 2]
            ).wait()

            xg = [xb[b][g % NXB] for b in range(B)]
            pg = pb[g % 2]

            @plsc.parallel_loop(0, E, step=L, unroll=1)
            def add_body(o):
                for r in range(C):
                    pv = pg[r, pl.ds(o, L)]
                    for b in range(B):
                        xg[b][r, pl.ds(o, L)] = xg[b][r, pl.ds(o, L)] + pv

            for b in range(B):
                j = g % NXB
                pltpu.async_copy(
                    xg[b], out_hbm.at[pl.ds(x_row(b, g), C)], sxb[b][j]
                )

        for g in (NG - 3, NG - 2, NG - 1):
            for b in range(B):
                wait_out(b, g)

    out = sc_add(xf, pos_table)
    return out.reshape(B, S, E)
